# baseline (speedup 1.0000x reference)
import functools

import jax
import jax.numpy as jnp
from jax import lax
from jax.experimental import pallas as pl
from jax.experimental.pallas import tpu as pltpu
from jax.experimental.pallas import tpu_sc as plsc

T, C, H, HD, E, FF = 2048, 768, 12, 64, 16, 3072
MT = 256
PTOT = 2 * T + E * MT
NT = PTOT // MT
NW = 32
TPW = T // NW
SUB = 32
TQ = 256


def _ln_qkv_body(x_ref, g_ref, b_ref, w_ref, qkv_ref):
    xb = x_ref[...]
    m = jnp.mean(xb, axis=1, keepdims=True)
    v = jnp.mean((xb - m) ** 2, axis=1, keepdims=True)
    h = (xb - m) / jnp.sqrt(v + 1e-5) * g_ref[...] + b_ref[...]
    qkv_ref[...] = jnp.dot(h, w_ref[...], preferred_element_type=jnp.float32)


def _attn_body(q_ref, k_ref, v_ref, o_ref):
    q = q_ref[0]
    k = k_ref[0]
    s = lax.dot_general(q, k, (((1,), (1,)), ((), ())),
                        preferred_element_type=jnp.float32) * (C ** -0.5)
    m = jnp.max(s, axis=1, keepdims=True)
    p = jnp.exp(s - m)
    p = p / jnp.sum(p, axis=1, keepdims=True)
    o_ref[0] = jnp.dot(p, v_ref[0], preferred_element_type=jnp.float32)


def _router_body(o_ref, wp_ref, bp_ref, x_ref, g2_ref, b2_ref, wr_ref, br_ref,
                 wn_ref, bn_ref, nz_ref, h2_ref, pos1_ref, pos2_ref,
                 gt1_ref, gt2_ref, te_ref):
    attn = jnp.dot(o_ref[...], wp_ref[...],
                   preferred_element_type=jnp.float32) + bp_ref[...]
    x2 = x_ref[...] + attn
    m = jnp.mean(x2, axis=1, keepdims=True)
    v = jnp.mean((x2 - m) ** 2, axis=1, keepdims=True)
    h2 = (x2 - m) / jnp.sqrt(v + 1e-5) * g2_ref[...] + b2_ref[...]
    h2_ref[...] = h2

    logits = jnp.dot(h2, wr_ref[...],
                     preferred_element_type=jnp.float32) + br_ref[...]
    nlog = jnp.dot(h2, wn_ref[...],
                   preferred_element_type=jnp.float32) + bn_ref[...]
    sp = jnp.maximum(nlog, 0.0) + jnp.log1p(jnp.exp(-jnp.abs(nlog)))
    noisy = logits + nz_ref[...] * sp

    eidx = lax.broadcasted_iota(jnp.int32, (T, E), 1)
    m1 = jnp.max(noisy, axis=1, keepdims=True)
    i1 = jnp.min(jnp.where(noisy == m1, eidx, E), axis=1, keepdims=True)
    n2 = jnp.where(eidx == i1, -jnp.inf, noisy)
    m2 = jnp.max(n2, axis=1, keepdims=True)
    i2 = jnp.min(jnp.where(n2 == m2, eidx, E), axis=1, keepdims=True)
    e21 = jnp.exp(m2 - m1)
    gt1_ref[...] = jnp.broadcast_to(1.0 / (1.0 + e21), (T, E))
    gt2_ref[...] = jnp.broadcast_to(e21 / (1.0 + e21), (T, E))

    msk = ((eidx == i1) | (eidx == i2)).astype(jnp.float32)
    csum = msk
    sh = 1
    while sh < T:
        csum = csum + jnp.concatenate(
            [jnp.zeros((sh, E), jnp.float32), csum[:T - sh]], axis=0)
        sh *= 2
    cexc = (csum - msk).astype(jnp.int32)
    ci = csum[T - 1:T, :].astype(jnp.int32)
    pc = ((ci + (MT - 1)) // MT) * MT
    oi = pc
    sh = 1
    while sh < E:
        oi = oi + jnp.concatenate(
            [jnp.zeros((1, sh), jnp.int32), oi[:, :E - sh]], axis=1)
        sh *= 2
    off = oi - pc
    pos = off + cexc
    pos1_ref[...] = jnp.sum(jnp.where(eidx == i1, pos, 0), axis=1,
                            keepdims=True)
    pos2_ref[...] = jnp.sum(jnp.where(eidx == i2, pos, 0), axis=1,
                            keepdims=True)

    erow = lax.broadcasted_iota(jnp.int32, (1, E), 1)
    la = jnp.max(jnp.where(ci > 0, erow, 0), axis=1, keepdims=True)
    jt = lax.broadcasted_iota(jnp.int32, (NT, 1), 0) * MT
    nfull = jnp.sum((jt >= oi).astype(jnp.int32), axis=1, keepdims=True)
    te_ref[...] = jnp.minimum(nfull, la)


def _expert_body(te_ref, xs_ref, w1_ref, b1_ref, w2_ref, b2_ref, ys_ref):
    del te_ref
    xb = xs_ref[...]
    a = jnp.dot(xb, w1_ref[0], preferred_element_type=jnp.float32) + b1_ref[0]
    a = jnp.maximum(a, 0.0)
    y = jnp.dot(a, w2_ref[0], preferred_element_type=jnp.float32) + b2_ref[0]
    ys_ref[...] = y


def _dispatch_body(h2_hbm, pos1_hbm, pos2_hbm, xs_hbm, rows_v, i1_v, i2_v,
                   sem):
    wid = lax.axis_index("s") * 2 + lax.axis_index("c")
    base = wid * TPW
    pltpu.sync_copy(h2_hbm.at[pl.ds(base, TPW)], rows_v)
    pltpu.sync_copy(pos1_hbm.at[pl.ds(base, TPW)], i1_v)
    pltpu.sync_copy(pos2_hbm.at[pl.ds(base, TPW)], i2_v)
    c1 = pltpu.async_copy(rows_v, xs_hbm.at[i1_v], sem)
    c2 = pltpu.async_copy(rows_v, xs_hbm.at[i2_v], sem)
    c1.wait()
    c2.wait()


def _combine_body(h2_hbm, ys_hbm, pos1_hbm, pos2_hbm, g1_hbm, g2_hbm, out_hbm,
                  acc_v, y1_v, y2_v, i1_v, i2_v, g1_v, g2_v, sem):
    wid = lax.axis_index("s") * 2 + lax.axis_index("c")
    for s in range(TPW // SUB):
        base = wid * TPW + s * SUB
        pltpu.sync_copy(h2_hbm.at[pl.ds(base, SUB)], acc_v)
        pltpu.sync_copy(pos1_hbm.at[pl.ds(base, SUB)], i1_v)
        pltpu.sync_copy(pos2_hbm.at[pl.ds(base, SUB)], i2_v)
        pltpu.sync_copy(g1_hbm.at[pl.ds(base, SUB)], g1_v)
        pltpu.sync_copy(g2_hbm.at[pl.ds(base, SUB)], g2_v)
        c1 = pltpu.async_copy(ys_hbm.at[i1_v], y1_v, sem)
        c2 = pltpu.async_copy(ys_hbm.at[i2_v], y2_v, sem)
        c1.wait()
        c2.wait()

        def tok(i, _):
            g1s = g1_v[i, :]
            g2s = g2_v[i, :]
            for cc in range(C // 16):
                sl = pl.ds(cc * 16, 16)
                acc_v[i, sl] = (acc_v[i, sl] + g1s * y1_v[i, sl]
                                + g2s * y2_v[i, sl])
            return 0

        lax.fori_loop(0, SUB, tok, 0)
        pltpu.sync_copy(acc_v, out_hbm.at[pl.ds(base, SUB)])


@functools.cache
def _sc_kernels():
    mesh = plsc.VectorSubcoreMesh(core_axis_name="c", subcore_axis_name="s")
    dispatch = pl.kernel(
        _dispatch_body,
        out_type=jax.ShapeDtypeStruct((PTOT, C), jnp.float32),
        mesh=mesh,
        scratch_types=[
            pltpu.VMEM((TPW, C), jnp.float32),
            pltpu.VMEM((TPW,), jnp.int32),
            pltpu.VMEM((TPW,), jnp.int32),
            pltpu.SemaphoreType.DMA,
        ],
    )
    combine = pl.kernel(
        _combine_body,
        out_type=jax.ShapeDtypeStruct((T, C), jnp.float32),
        mesh=mesh,
        scratch_types=[
            pltpu.VMEM((SUB, C), jnp.float32),
            pltpu.VMEM((SUB, C), jnp.float32),
            pltpu.VMEM((SUB, C), jnp.float32),
            pltpu.VMEM((SUB,), jnp.int32),
            pltpu.VMEM((SUB,), jnp.int32),
            pltpu.VMEM((SUB, E), jnp.float32),
            pltpu.VMEM((SUB, E), jnp.float32),
            pltpu.SemaphoreType.DMA,
        ],
    )
    return dispatch, combine


def kernel(x, noise_std, gamma1, beta1, Wq, Wk, Wv, Wproj, bproj, gamma2,
           beta2, Wr, br, Wn, bn, We1, be1, We2, be2):
    f32 = jnp.float32
    x2d = x.reshape(T, C)
    nz = noise_std.reshape(T, E)
    wqkv = jnp.concatenate(
        [Wq.transpose(1, 0, 2).reshape(C, C),
         Wk.transpose(1, 0, 2).reshape(C, C),
         Wv.transpose(1, 0, 2).reshape(C, C)], axis=1)

    qkv = pl.pallas_call(
        _ln_qkv_body,
        grid=(T // TQ,),
        in_specs=[
            pl.BlockSpec((TQ, C), lambda i: (i, 0)),
            pl.BlockSpec((1, C), lambda i: (0, 0)),
            pl.BlockSpec((1, C), lambda i: (0, 0)),
            pl.BlockSpec((C, 3 * C), lambda i: (0, 0)),
        ],
        out_specs=pl.BlockSpec((TQ, 3 * C), lambda i: (i, 0)),
        out_shape=jax.ShapeDtypeStruct((T, 3 * C), f32),
    )(x2d, gamma1.reshape(1, C), beta1.reshape(1, C), wqkv)

    q3 = qkv[:, :C].reshape(T, H, HD).transpose(1, 0, 2)
    k3 = qkv[:, C:2 * C].reshape(T, H, HD).transpose(1, 0, 2)
    v3 = qkv[:, 2 * C:].reshape(T, H, HD).transpose(1, 0, 2)
    o3 = pl.pallas_call(
        _attn_body,
        grid=(H, T // TQ),
        in_specs=[
            pl.BlockSpec((1, TQ, HD), lambda h, i: (h, i, 0)),
            pl.BlockSpec((1, T, HD), lambda h, i: (h, 0, 0)),
            pl.BlockSpec((1, T, HD), lambda h, i: (h, 0, 0)),
        ],
        out_specs=pl.BlockSpec((1, TQ, HD), lambda h, i: (h, i, 0)),
        out_shape=jax.ShapeDtypeStruct((H, T, HD), f32),
    )(q3, k3, v3)
    o = o3.transpose(1, 0, 2).reshape(T, C)

    h2, pos1, pos2, gt1, gt2, te = pl.pallas_call(
        _router_body,
        out_shape=[
            jax.ShapeDtypeStruct((T, C), f32),
            jax.ShapeDtypeStruct((T, 1), jnp.int32),
            jax.ShapeDtypeStruct((T, 1), jnp.int32),
            jax.ShapeDtypeStruct((T, E), f32),
            jax.ShapeDtypeStruct((T, E), f32),
            jax.ShapeDtypeStruct((NT, 1), jnp.int32),
        ],
    )(o, Wproj, bproj.reshape(1, C), x2d, gamma2.reshape(1, C),
      beta2.reshape(1, C), Wr, br.reshape(1, E), Wn, bn.reshape(1, E), nz)

    p1 = pos1.reshape(T)
    p2 = pos2.reshape(T)
    _dispatch, _combine = _sc_kernels()
    xs = _dispatch(h2, p1, p2)

    ys = pl.pallas_call(
        _expert_body,
        grid_spec=pltpu.PrefetchScalarGridSpec(
            num_scalar_prefetch=1,
            grid=(NT,),
            in_specs=[
                pl.BlockSpec((MT, C), lambda j, te: (j, 0)),
                pl.BlockSpec((1, C, FF), lambda j, te: (te[j], 0, 0)),
                pl.BlockSpec((1, 1, FF), lambda j, te: (te[j], 0, 0)),
                pl.BlockSpec((1, FF, C), lambda j, te: (te[j], 0, 0)),
                pl.BlockSpec((1, 1, C), lambda j, te: (te[j], 0, 0)),
            ],
            out_specs=pl.BlockSpec((MT, C), lambda j, te: (j, 0)),
        ),
        out_shape=jax.ShapeDtypeStruct((PTOT, C), f32),
    )(te.reshape(NT), xs, We1, be1.reshape(E, 1, FF), We2,
      be2.reshape(E, 1, C))

    out = _combine(h2, ys, p1, p2, gt1, gt2)
    return out.reshape(1, T, C)


# speedup vs baseline: 1.2610x; 1.2610x over previous
import functools

import jax
import jax.numpy as jnp
from jax import lax
from jax.experimental import pallas as pl
from jax.experimental.pallas import tpu as pltpu
from jax.experimental.pallas import tpu_sc as plsc

T, C, H, HD, E, FF = 2048, 768, 12, 64, 16, 3072
MT = 256
PTOT = 2 * T + E * MT
NT = PTOT // MT
NW = 32
TPW = T // NW
SUB = 32
TQ = 256


def _ln_qkv_body(x_ref, g_ref, b_ref, w_ref, qkv_ref):
    xb = x_ref[...]
    m = jnp.mean(xb, axis=1, keepdims=True)
    v = jnp.mean((xb - m) ** 2, axis=1, keepdims=True)
    h = (xb - m) / jnp.sqrt(v + 1e-5) * g_ref[...] + b_ref[...]
    qkv_ref[...] = jnp.dot(h, w_ref[...], preferred_element_type=jnp.float32)


def _attn_body(q_ref, k_ref, v_ref, o_ref):
    qp = q_ref[...].astype(jnp.bfloat16)
    kp = k_ref[...].astype(jnp.bfloat16)
    vp = v_ref[...].astype(jnp.bfloat16)
    outs = []
    for hh in range(2):
        q = qp[:, hh * HD:(hh + 1) * HD]
        k = kp[:, hh * HD:(hh + 1) * HD]
        v = vp[:, hh * HD:(hh + 1) * HD]
        s = lax.dot_general(q, k, (((1,), (1,)), ((), ())),
                            preferred_element_type=jnp.float32) * (C ** -0.5)
        m = jnp.max(s, axis=1, keepdims=True)
        p = jnp.exp(s - m)
        p = (p / jnp.sum(p, axis=1, keepdims=True)).astype(jnp.bfloat16)
        outs.append(jnp.dot(p, v, preferred_element_type=jnp.float32))
    o_ref[...] = jnp.concatenate(outs, axis=1)


def _router_body(o_ref, wp_ref, bp_ref, x_ref, g2_ref, b2_ref, wr_ref, br_ref,
                 wn_ref, bn_ref, nz_ref, h2_ref, pos1_ref, pos2_ref,
                 gt1_ref, gt2_ref, te_ref):
    attn = jnp.dot(o_ref[...], wp_ref[...],
                   preferred_element_type=jnp.float32) + bp_ref[...]
    x2 = x_ref[...] + attn
    m = jnp.mean(x2, axis=1, keepdims=True)
    v = jnp.mean((x2 - m) ** 2, axis=1, keepdims=True)
    h2 = (x2 - m) / jnp.sqrt(v + 1e-5) * g2_ref[...] + b2_ref[...]
    h2_ref[...] = h2

    logits = jnp.dot(h2, wr_ref[...],
                     preferred_element_type=jnp.float32) + br_ref[...]
    nlog = jnp.dot(h2, wn_ref[...],
                   preferred_element_type=jnp.float32) + bn_ref[...]
    sp = jnp.maximum(nlog, 0.0) + jnp.log1p(jnp.exp(-jnp.abs(nlog)))
    noisy = logits + nz_ref[...] * sp

    eidx = lax.broadcasted_iota(jnp.int32, (T, E), 1)
    m1 = jnp.max(noisy, axis=1, keepdims=True)
    i1 = jnp.min(jnp.where(noisy == m1, eidx, E), axis=1, keepdims=True)
    n2 = jnp.where(eidx == i1, -jnp.inf, noisy)
    m2 = jnp.max(n2, axis=1, keepdims=True)
    i2 = jnp.min(jnp.where(n2 == m2, eidx, E), axis=1, keepdims=True)
    e21 = jnp.exp(m2 - m1)
    gt1_ref[...] = jnp.broadcast_to(1.0 / (1.0 + e21), (T, E))
    gt2_ref[...] = jnp.broadcast_to(e21 / (1.0 + e21), (T, E))

    msk = ((eidx == i1) | (eidx == i2)).astype(jnp.float32)
    csum = msk
    sh = 1
    while sh < T:
        csum = csum + jnp.concatenate(
            [jnp.zeros((sh, E), jnp.float32), csum[:T - sh]], axis=0)
        sh *= 2
    cexc = (csum - msk).astype(jnp.int32)
    ci = csum[T - 1:T, :].astype(jnp.int32)
    pc = ((ci + (MT - 1)) // MT) * MT
    oi = pc
    sh = 1
    while sh < E:
        oi = oi + jnp.concatenate(
            [jnp.zeros((1, sh), jnp.int32), oi[:, :E - sh]], axis=1)
        sh *= 2
    off = oi - pc
    pos = off + cexc
    pos1_ref[...] = jnp.sum(jnp.where(eidx == i1, pos, 0), axis=1,
                            keepdims=True)
    pos2_ref[...] = jnp.sum(jnp.where(eidx == i2, pos, 0), axis=1,
                            keepdims=True)

    erow = lax.broadcasted_iota(jnp.int32, (1, E), 1)
    la = jnp.max(jnp.where(ci > 0, erow, 0), axis=1, keepdims=True)
    jt = lax.broadcasted_iota(jnp.int32, (NT, 1), 0) * MT
    nfull = jnp.sum((jt >= oi).astype(jnp.int32), axis=1, keepdims=True)
    te_ref[...] = jnp.minimum(nfull, la)


def _expert_body(te_ref, xs_ref, w1_ref, b1_ref, w2_ref, b2_ref, ys_ref):
    del te_ref
    xb = xs_ref[...]
    a = jnp.dot(xb, w1_ref[0], preferred_element_type=jnp.float32) + b1_ref[0]
    a = jnp.maximum(a, 0.0)
    y = jnp.dot(a, w2_ref[0], preferred_element_type=jnp.float32) + b2_ref[0]
    ys_ref[...] = y


def _dispatch_body(h2_hbm, pos1_hbm, pos2_hbm, xs_hbm, rows_v, i1_v, i2_v,
                   sem):
    wid = lax.axis_index("s") * 2 + lax.axis_index("c")
    base = wid * TPW
    pltpu.sync_copy(h2_hbm.at[pl.ds(base, TPW)], rows_v)
    pltpu.sync_copy(pos1_hbm.at[pl.ds(base, TPW)], i1_v)
    pltpu.sync_copy(pos2_hbm.at[pl.ds(base, TPW)], i2_v)
    c1 = pltpu.async_copy(rows_v, xs_hbm.at[i1_v], sem)
    c2 = pltpu.async_copy(rows_v, xs_hbm.at[i2_v], sem)
    c1.wait()
    c2.wait()


def _combine_body(h2_hbm, ys_hbm, pos1_hbm, pos2_hbm, g1_hbm, g2_hbm, out_hbm,
                  acc_v, y1_v, y2_v, i1_v, i2_v, g1_v, g2_v, sem):
    wid = lax.axis_index("s") * 2 + lax.axis_index("c")
    for s in range(TPW // SUB):
        base = wid * TPW + s * SUB
        pltpu.sync_copy(h2_hbm.at[pl.ds(base, SUB)], acc_v)
        pltpu.sync_copy(pos1_hbm.at[pl.ds(base, SUB)], i1_v)
        pltpu.sync_copy(pos2_hbm.at[pl.ds(base, SUB)], i2_v)
        pltpu.sync_copy(g1_hbm.at[pl.ds(base, SUB)], g1_v)
        pltpu.sync_copy(g2_hbm.at[pl.ds(base, SUB)], g2_v)
        c1 = pltpu.async_copy(ys_hbm.at[i1_v], y1_v, sem)
        c2 = pltpu.async_copy(ys_hbm.at[i2_v], y2_v, sem)
        c1.wait()
        c2.wait()

        def tok(i, _):
            g1s = g1_v[i, :]
            g2s = g2_v[i, :]
            for cc in range(C // 16):
                sl = pl.ds(cc * 16, 16)
                acc_v[i, sl] = (acc_v[i, sl] + g1s * y1_v[i, sl]
                                + g2s * y2_v[i, sl])
            return 0

        lax.fori_loop(0, SUB, tok, 0)
        pltpu.sync_copy(acc_v, out_hbm.at[pl.ds(base, SUB)])


@functools.cache
def _sc_kernels():
    mesh = plsc.VectorSubcoreMesh(core_axis_name="c", subcore_axis_name="s")
    dispatch = pl.kernel(
        _dispatch_body,
        out_type=jax.ShapeDtypeStruct((PTOT, C), jnp.float32),
        mesh=mesh,
        scratch_types=[
            pltpu.VMEM((TPW, C), jnp.float32),
            pltpu.VMEM((TPW,), jnp.int32),
            pltpu.VMEM((TPW,), jnp.int32),
            pltpu.SemaphoreType.DMA,
        ],
    )
    combine = pl.kernel(
        _combine_body,
        out_type=jax.ShapeDtypeStruct((T, C), jnp.float32),
        mesh=mesh,
        scratch_types=[
            pltpu.VMEM((SUB, C), jnp.float32),
            pltpu.VMEM((SUB, C), jnp.float32),
            pltpu.VMEM((SUB, C), jnp.float32),
            pltpu.VMEM((SUB,), jnp.int32),
            pltpu.VMEM((SUB,), jnp.int32),
            pltpu.VMEM((SUB, E), jnp.float32),
            pltpu.VMEM((SUB, E), jnp.float32),
            pltpu.SemaphoreType.DMA,
        ],
    )
    return dispatch, combine


def kernel(x, noise_std, gamma1, beta1, Wq, Wk, Wv, Wproj, bproj, gamma2,
           beta2, Wr, br, Wn, bn, We1, be1, We2, be2):
    f32 = jnp.float32
    x2d = x.reshape(T, C)
    nz = noise_std.reshape(T, E)
    wqkv = jnp.concatenate(
        [Wq.transpose(1, 0, 2).reshape(C, C),
         Wk.transpose(1, 0, 2).reshape(C, C),
         Wv.transpose(1, 0, 2).reshape(C, C)], axis=1)

    qkv = pl.pallas_call(
        _ln_qkv_body,
        grid=(T // TQ,),
        in_specs=[
            pl.BlockSpec((TQ, C), lambda i: (i, 0)),
            pl.BlockSpec((1, C), lambda i: (0, 0)),
            pl.BlockSpec((1, C), lambda i: (0, 0)),
            pl.BlockSpec((C, 3 * C), lambda i: (0, 0)),
        ],
        out_specs=pl.BlockSpec((TQ, 3 * C), lambda i: (i, 0)),
        out_shape=jax.ShapeDtypeStruct((T, 3 * C), f32),
    )(x2d, gamma1.reshape(1, C), beta1.reshape(1, C), wqkv)

    o = pl.pallas_call(
        _attn_body,
        grid=(H // 2, T // TQ),
        in_specs=[
            pl.BlockSpec((TQ, 2 * HD), lambda hh, i: (i, hh)),
            pl.BlockSpec((T, 2 * HD), lambda hh, i: (0, H // 2 + hh)),
            pl.BlockSpec((T, 2 * HD), lambda hh, i: (0, H + hh)),
        ],
        out_specs=pl.BlockSpec((TQ, 2 * HD), lambda hh, i: (i, hh)),
        out_shape=jax.ShapeDtypeStruct((T, C), f32),
    )(qkv, qkv, qkv)

    h2, pos1, pos2, gt1, gt2, te = pl.pallas_call(
        _router_body,
        out_shape=[
            jax.ShapeDtypeStruct((T, C), f32),
            jax.ShapeDtypeStruct((T, 1), jnp.int32),
            jax.ShapeDtypeStruct((T, 1), jnp.int32),
            jax.ShapeDtypeStruct((T, E), f32),
            jax.ShapeDtypeStruct((T, E), f32),
            jax.ShapeDtypeStruct((NT, 1), jnp.int32),
        ],
    )(o, Wproj, bproj.reshape(1, C), x2d, gamma2.reshape(1, C),
      beta2.reshape(1, C), Wr, br.reshape(1, E), Wn, bn.reshape(1, E), nz)

    p1 = pos1.reshape(T)
    p2 = pos2.reshape(T)
    _dispatch, _combine = _sc_kernels()
    xs = _dispatch(h2, p1, p2)

    ys = pl.pallas_call(
        _expert_body,
        grid_spec=pltpu.PrefetchScalarGridSpec(
            num_scalar_prefetch=1,
            grid=(NT,),
            in_specs=[
                pl.BlockSpec((MT, C), lambda j, te: (j, 0)),
                pl.BlockSpec((1, C, FF), lambda j, te: (te[j], 0, 0)),
                pl.BlockSpec((1, 1, FF), lambda j, te: (te[j], 0, 0)),
                pl.BlockSpec((1, FF, C), lambda j, te: (te[j], 0, 0)),
                pl.BlockSpec((1, 1, C), lambda j, te: (te[j], 0, 0)),
            ],
            out_specs=pl.BlockSpec((MT, C), lambda j, te: (j, 0)),
        ),
        out_shape=jax.ShapeDtypeStruct((PTOT, C), f32),
    )(te.reshape(NT), xs, We1, be1.reshape(E, 1, FF), We2,
      be2.reshape(E, 1, C))

    out = _combine(h2, ys, p1, p2, gt1, gt2)
    return out.reshape(1, T, C)
